# baseline (speedup 1.0000x reference)
import functools

import jax
import jax.numpy as jnp
from jax import lax
from jax.experimental import pallas as pl
from jax.experimental.pallas import tpu as pltpu
from jax.experimental.pallas import tpu_sc as plsc

N = 10000
E = 160000
D = 128
NC = 2
NS = 16
CL = 80
CH = 64
CL1 = 128
CH1 = 40
EPT = CH * CL
EP = NC * NS * EPT
NP = 10240
RPT = NP // NS
KB = RPT // CL

_mesh = plsc.VectorSubcoreMesh(core_axis_name="c", subcore_axis_name="s")


NB1 = 4


@functools.partial(
    pl.kernel,
    mesh=_mesh,
    out_type=jax.ShapeDtypeStruct((NC, 6, NP), jnp.float32),
    scratch_types=[
        pltpu.VMEM((CH1, CL1), jnp.int32),
        pltpu.VMEM((CL1,), jnp.float32),
        pltpu.VMEM((RPT,), jnp.float32),
    ] + [pltpu.VMEM_SHARED((NP,), jnp.float32) for _ in range(6)]
      + [pltpu.SemaphoreType.DMA for _ in range(NB1)],
)
def _k1_degrees(i0, i1, i2, i3, i4, i5, ones_hbm, zeros1_hbm, degp_hbm,
                idx_v, ones_v, buf_v, h0, h1, h2, h3, h4, h5, s0, s1, s2, s3):
    cid = lax.axis_index("c")
    sid = lax.axis_index("s")
    hists = (h0, h1, h2, h3, h4, h5)
    idxs = (i0, i1, i2, i3, i4, i5)
    sems = (s0, s1, s2, s3)
    pltpu.sync_copy(ones_hbm, ones_v)
    pltpu.sync_copy(zeros1_hbm, buf_v)
    for h in hists:
        pltpu.sync_copy(buf_v, h.at[pl.ds(sid * RPT, RPT)])
    plsc.subcore_barrier()
    for a, (h, ix) in enumerate(zip(hists, idxs)):
        pltpu.sync_copy(ix.at[cid, sid], idx_v)
        for b in range(NB1 - 1):
            pltpu.async_copy(ones_v, h.at[idx_v.at[b]], sems[b], add=True)

        def step(t, carry, h=h):
            for b in range(NB1):
                j = t * NB1 + b
                jn = j + NB1 - 1
                fs = (b + NB1 - 1) % NB1

                @pl.when(jn < CH1)
                def _(jn=jn, fs=fs, h=h):
                    pltpu.async_copy(ones_v, h.at[idx_v.at[jn]], sems[fs],
                                     add=True)

                pltpu.make_async_copy(ones_v, h.at[idx_v.at[j]],
                                      sems[b]).wait()
            return carry

        lax.fori_loop(0, CH1 // NB1, step, 0)
    plsc.subcore_barrier()
    for a, h in enumerate(hists):
        pltpu.sync_copy(h.at[pl.ds(sid * RPT, RPT)], buf_v)
        pltpu.sync_copy(buf_v, degp_hbm.at[cid, a, pl.ds(sid * RPT, RPT)])


_BM = 2000


def _k2_body(deg_ref, corr_ref, xu_ref, xi_ref, wf_ref, wrb_ref, wrt_ref,
             yf_ref, yrb_ref, yrt_ref):
    def scaled(x_ref, slot):
        d = deg_ref[:, slot] + deg_ref[:, 6 + slot] - corr_ref[:, 0]
        ns = jnp.where(d > 0, lax.rsqrt(jnp.maximum(d, 1.0)), 0.0)
        return x_ref[...] * ns[:, None]

    yf_ref[...] = jnp.dot(scaled(xu_ref, 0), wf_ref[...],
                          preferred_element_type=jnp.float32)
    yrb_ref[...] = jnp.dot(scaled(xi_ref, 2), wrb_ref[...],
                           preferred_element_type=jnp.float32)
    yrt_ref[...] = jnp.dot(scaled(xu_ref, 4), wrt_ref[...],
                           preferred_element_type=jnp.float32)


def _k2_transform(degp, corr, xu, xi, w_f, w_rb, w_rt):
    y_shape = jax.ShapeDtypeStruct((N, D), jnp.float32)
    return pl.pallas_call(
        _k2_body,
        grid=(N // _BM,),
        in_specs=[
            pl.BlockSpec((_BM, 2 * 6), lambda i: (i, 0)),
            pl.BlockSpec((_BM, 1), lambda i: (i, 0)),
            pl.BlockSpec((_BM, D), lambda i: (i, 0)),
            pl.BlockSpec((_BM, D), lambda i: (i, 0)),
            pl.BlockSpec((D, D), lambda i: (0, 0)),
            pl.BlockSpec((D, D), lambda i: (0, 0)),
            pl.BlockSpec((D, D), lambda i: (0, 0)),
        ],
        out_specs=[
            pl.BlockSpec((_BM, D), lambda i: (i, 0)),
            pl.BlockSpec((_BM, D), lambda i: (i, 0)),
            pl.BlockSpec((_BM, D), lambda i: (i, 0)),
        ],
        out_shape=[y_shape, y_shape, y_shape],
    )(degp, corr, xu, xi, w_f, w_rb, w_rt)


NSL = 3
_part_t = jax.ShapeDtypeStruct((NC, NP, D), jnp.float32)


@functools.partial(
    pl.kernel,
    mesh=_mesh,
    out_type=(_part_t, _part_t, _part_t),
    scratch_types=[
        pltpu.VMEM_SHARED((NP, D), jnp.float32),
        pltpu.VMEM((CH, CL), jnp.int32),
        pltpu.VMEM((CH, CL), jnp.int32),
        pltpu.VMEM((NSL, CL, D), jnp.float32),
    ] + [pltpu.SemaphoreType.DMA for _ in range(2 * NSL)],
)
def _k3_scatter(yf_hbm, yrb_hbm, yrt_hbm, sf_hbm, df_hbm, srb_hbm, drb_hbm,
                srt_hbm, drt_hbm, zeros2_hbm, pf_hbm, prb_hbm, prt_hbm,
                acc_s, sidx_v, didx_v, rows_v, g0, g1, g2, t0, t1, t2):
    cid = lax.axis_index("c")
    sid = lax.axis_index("s")
    gsems = (g0, g1, g2)
    ssems = (t0, t1, t2)
    for y_hbm, s_hbm, d_hbm, p_hbm in (
            (yf_hbm, sf_hbm, df_hbm, pf_hbm),
            (yrb_hbm, srb_hbm, drb_hbm, prb_hbm),
            (yrt_hbm, srt_hbm, drt_hbm, prt_hbm)):
        pltpu.sync_copy(zeros2_hbm, rows_v.at[NSL - 1])
        for z in range(KB):
            pltpu.async_copy(rows_v.at[NSL - 1],
                             acc_s.at[pl.ds(sid * RPT + z * CL, CL), :],
                             ssems[0])
        pltpu.sync_copy(s_hbm.at[cid, sid], sidx_v)
        pltpu.sync_copy(d_hbm.at[cid, sid], didx_v)

        def gfire(jn, bn, y_hbm=y_hbm):
            pltpu.async_copy(y_hbm.at[sidx_v.at[jn]], rows_v.at[bn],
                             gsems[bn])

        def gwait(j, b, y_hbm=y_hbm):
            pltpu.make_async_copy(y_hbm.at[sidx_v.at[j]], rows_v.at[b],
                                  gsems[b]).wait()

        def sfire(j, b):
            pltpu.async_copy(rows_v.at[b], acc_s.at[didx_v.at[j]], ssems[b],
                             add=True)

        def swait(j, b):
            pltpu.make_async_copy(rows_v.at[b], acc_s.at[didx_v.at[j]],
                                  ssems[b]).wait()

        gfire(0, 0)
        gfire(1, 1)
        for z in range(KB):
            pltpu.make_async_copy(
                rows_v.at[NSL - 1], acc_s.at[pl.ds(sid * RPT + z * CL, CL), :],
                ssems[0]).wait()
        plsc.subcore_barrier()
        gwait(0, 0)
        sfire(0, 0)
        gfire(2, 2)

        def step(t, carry, y_hbm=y_hbm):
            for i in range(NSL):
                j = 1 + t * NSL + i
                b = (1 + i) % NSL
                gwait(j, b)
                sfire(j, b)
                jn = j + 2
                bn = i % NSL

                @pl.when(jn < CH)
                def _(jn=jn, bn=bn):
                    swait(jn - NSL, bn)
                    gfire(jn, bn)
            return carry

        lax.fori_loop(0, (CH - 1) // NSL, step, 0)
        for j in range(CH - NSL, CH):
            swait(j, j % NSL)
        plsc.subcore_barrier()
        for k in range(KB):
            sl = k % NSL
            if k >= NSL:
                rp = sid * RPT + (k - NSL) * CL
                pltpu.make_async_copy(rows_v.at[sl],
                                      p_hbm.at[cid, pl.ds(rp, CL), :],
                                      gsems[sl]).wait()
            r0 = sid * RPT + k * CL
            pltpu.sync_copy(acc_s.at[pl.ds(r0, CL), :], rows_v.at[sl])
            pltpu.async_copy(rows_v.at[sl], p_hbm.at[cid, pl.ds(r0, CL), :],
                             gsems[sl])
        for k in range(KB - NSL, KB):
            sl = k % NSL
            r0 = sid * RPT + k * CL
            pltpu.make_async_copy(rows_v.at[sl],
                                  p_hbm.at[cid, pl.ds(r0, CL), :],
                                  gsems[sl]).wait()


def _k4_body(deg_ref, pf_ref, prb_ref, prt_ref, bf_ref, brb_ref, brt_ref,
             hu_ref, hi_ref):
    def nd(slot):
        d = deg_ref[:, slot] + deg_ref[:, 6 + slot]
        return jnp.where(d > 0, lax.rsqrt(jnp.maximum(d, 1.0)), 0.0)

    agg_f = (pf_ref[0] + pf_ref[1]) * nd(1)[:, None] + bf_ref[...][None, :]
    agg_rb = (prb_ref[0] + prb_ref[1]) * nd(3)[:, None] + brb_ref[...][None, :]
    hu_ref[...] = agg_f + agg_rb
    hi_ref[...] = ((prt_ref[0] + prt_ref[1]) * nd(5)[:, None]
                   + brt_ref[...][None, :])


def _k4_combine(degp, p_f, p_rb, p_rt, b_f, b_rb, b_rt):
    h_shape = jax.ShapeDtypeStruct((N, D), jnp.float32)
    part_spec = pl.BlockSpec((NC, _BM, D), lambda i: (0, i, 0))
    bias_spec = pl.BlockSpec((D,), lambda i: (0,))
    return pl.pallas_call(
        _k4_body,
        grid=(N // _BM,),
        in_specs=[
            pl.BlockSpec((_BM, 2 * 6), lambda i: (i, 0)),
            part_spec, part_spec, part_spec,
            bias_spec, bias_spec, bias_spec,
        ],
        out_specs=[
            pl.BlockSpec((_BM, D), lambda i: (i, 0)),
            pl.BlockSpec((_BM, D), lambda i: (i, 0)),
        ],
        out_shape=[h_shape, h_shape],
    )(degp, p_f, p_rb, p_rt, b_f, b_rb, b_rt)


def _pad_src(vec):
    pad = jnp.arange(EP - E, dtype=jnp.int32) % N
    return jnp.concatenate([vec, pad]).reshape(NC, NS, CH, CL)


def _pad_dst(vec):
    pad = N + (jnp.arange(EP - E, dtype=jnp.int32) % (NP - N))
    return jnp.concatenate([vec, pad]).reshape(NC, NS, CH, CL)


def kernel(x_user, x_item, edge_index_follows, edge_index_rates,
           edge_index_rated_by, W_follows, b_follows, W_rates, b_rates,
           W_rated_by, b_rated_by):
    s_f = _pad_src(edge_index_follows[0])
    d_f = _pad_dst(edge_index_follows[1])
    s_rb = _pad_src(edge_index_rated_by[0])
    d_rb = _pad_dst(edge_index_rated_by[1])
    s_rt = _pad_src(edge_index_rates[0])
    d_rt = _pad_dst(edge_index_rates[1])

    ones128 = jnp.ones((CL1,), jnp.float32)
    zeros1 = jnp.zeros((RPT,), jnp.float32)
    zeros2 = jnp.zeros((CL, D), jnp.float32)

    def r1(a):
        return a.reshape(NC, NS, CH1, CL1)

    degp = _k1_degrees(r1(s_f), r1(d_f), r1(s_rb), r1(d_rb), r1(s_rt),
                       r1(d_rt), ones128, zeros1)
    degt = jnp.swapaxes(degp.reshape(NC * 6, NP), 0, 1)

    corr = (jnp.zeros((N,), jnp.float32)
            .at[jnp.arange(EP - E, dtype=jnp.int32) % N].add(1.0))[:, None]
    y_f, y_rb, y_rt = _k2_transform(degt, corr, x_user, x_item, W_follows,
                                    W_rated_by, W_rates)

    p_f, p_rb, p_rt = _k3_scatter(y_f, y_rb, y_rt, s_f, d_f, s_rb, d_rb,
                                  s_rt, d_rt, zeros2)

    return _k4_combine(degt, p_f, p_rb, p_rt, b_follows, b_rated_by, b_rates)


# speedup vs baseline: 1.2099x; 1.2099x over previous
import functools

import jax
import jax.numpy as jnp
from jax import lax
from jax.experimental import pallas as pl
from jax.experimental.pallas import tpu as pltpu
from jax.experimental.pallas import tpu_sc as plsc

N = 10000
E = 160000
D = 128
NC = 2
NS = 16
CL = 80
CH = 64
CL1 = 128
CH1 = 40
EPT = CH * CL
EP = NC * NS * EPT
NP = 10240
RPT = NP // NS
KB = RPT // CL

_mesh = plsc.VectorSubcoreMesh(core_axis_name="c", subcore_axis_name="s")


NB1 = 4


@functools.partial(
    pl.kernel,
    mesh=_mesh,
    out_type=jax.ShapeDtypeStruct((NC, 6, NP), jnp.float32),
    scratch_types=[
        pltpu.VMEM((CH1, CL1), jnp.int32),
        pltpu.VMEM((CL1,), jnp.float32),
        pltpu.VMEM((RPT,), jnp.float32),
    ] + [pltpu.VMEM_SHARED((NP,), jnp.float32) for _ in range(6)]
      + [pltpu.SemaphoreType.DMA for _ in range(NB1)],
)
def _k1_degrees(i0, i1, i2, i3, i4, i5, ones_hbm, zeros1_hbm, degp_hbm,
                idx_v, ones_v, buf_v, h0, h1, h2, h3, h4, h5, s0, s1, s2, s3):
    cid = lax.axis_index("c")
    sid = lax.axis_index("s")
    hists = (h0, h1, h2, h3, h4, h5)
    idxs = (i0, i1, i2, i3, i4, i5)
    sems = (s0, s1, s2, s3)
    pltpu.sync_copy(ones_hbm, ones_v)
    pltpu.sync_copy(zeros1_hbm, buf_v)
    for h in hists:
        pltpu.sync_copy(buf_v, h.at[pl.ds(sid * RPT, RPT)])
    plsc.subcore_barrier()
    for a, (h, ix) in enumerate(zip(hists, idxs)):
        pltpu.sync_copy(ix.at[cid, sid], idx_v)
        for b in range(NB1 - 1):
            pltpu.async_copy(ones_v, h.at[idx_v.at[b]], sems[b], add=True)

        def step(t, carry, h=h):
            for b in range(NB1):
                j = t * NB1 + b
                jn = j + NB1 - 1
                fs = (b + NB1 - 1) % NB1

                @pl.when(jn < CH1)
                def _(jn=jn, fs=fs, h=h):
                    pltpu.async_copy(ones_v, h.at[idx_v.at[jn]], sems[fs],
                                     add=True)

                pltpu.make_async_copy(ones_v, h.at[idx_v.at[j]],
                                      sems[b]).wait()
            return carry

        lax.fori_loop(0, CH1 // NB1, step, 0)
    plsc.subcore_barrier()
    for a, h in enumerate(hists):
        pltpu.sync_copy(h.at[pl.ds(sid * RPT, RPT)], buf_v)
        pltpu.sync_copy(buf_v, degp_hbm.at[cid, a, pl.ds(sid * RPT, RPT)])


_BM = 2000


def _k2_body(deg_ref, corr_ref, xu_ref, xi_ref, wf_ref, wrb_ref, wrt_ref,
             yf_ref, yrb_ref, yrt_ref):
    def scaled(x_ref, slot):
        d = deg_ref[:, slot] + deg_ref[:, 6 + slot] - corr_ref[:, 0]
        ns = jnp.where(d > 0, lax.rsqrt(jnp.maximum(d, 1.0)), 0.0)
        return x_ref[...] * ns[:, None]

    yf_ref[...] = jnp.dot(scaled(xu_ref, 0), wf_ref[...],
                          preferred_element_type=jnp.float32)
    yrb_ref[...] = jnp.dot(scaled(xi_ref, 2), wrb_ref[...],
                           preferred_element_type=jnp.float32)
    yrt_ref[...] = jnp.dot(scaled(xu_ref, 4), wrt_ref[...],
                           preferred_element_type=jnp.float32)


def _k2_transform(degp, corr, xu, xi, w_f, w_rb, w_rt):
    y_shape = jax.ShapeDtypeStruct((N, D), jnp.float32)
    return pl.pallas_call(
        _k2_body,
        grid=(N // _BM,),
        in_specs=[
            pl.BlockSpec((_BM, 2 * 6), lambda i: (i, 0)),
            pl.BlockSpec((_BM, 1), lambda i: (i, 0)),
            pl.BlockSpec((_BM, D), lambda i: (i, 0)),
            pl.BlockSpec((_BM, D), lambda i: (i, 0)),
            pl.BlockSpec((D, D), lambda i: (0, 0)),
            pl.BlockSpec((D, D), lambda i: (0, 0)),
            pl.BlockSpec((D, D), lambda i: (0, 0)),
        ],
        out_specs=[
            pl.BlockSpec((_BM, D), lambda i: (i, 0)),
            pl.BlockSpec((_BM, D), lambda i: (i, 0)),
            pl.BlockSpec((_BM, D), lambda i: (i, 0)),
        ],
        out_shape=[y_shape, y_shape, y_shape],
    )(degp, corr, xu, xi, w_f, w_rb, w_rt)


NSL = 3
_part_t = jax.ShapeDtypeStruct((NC, NP, D), jnp.float32)


@functools.partial(
    pl.kernel,
    mesh=_mesh,
    out_type=(_part_t, _part_t, _part_t),
    scratch_types=[
        pltpu.VMEM_SHARED((NP, D), jnp.float32),
        pltpu.VMEM((CH, CL), jnp.int32),
        pltpu.VMEM((CH, CL), jnp.int32),
        pltpu.VMEM((NSL, CL, D), jnp.float32),
    ] + [pltpu.SemaphoreType.DMA for _ in range(2 * NSL)],
)
def _k3_scatter(yf_hbm, yrb_hbm, yrt_hbm, sf_hbm, df_hbm, srb_hbm, drb_hbm,
                srt_hbm, drt_hbm, zeros2_hbm, pf_hbm, prb_hbm, prt_hbm,
                acc_s, sidx_v, didx_v, rows_v, g0, g1, g2, t0, t1, t2):
    cid = lax.axis_index("c")
    sid = lax.axis_index("s")
    gsems = (g0, g1, g2)
    ssems = (t0, t1, t2)
    for y_hbm, s_hbm, d_hbm, p_hbm in (
            (yf_hbm, sf_hbm, df_hbm, pf_hbm),
            (yrb_hbm, srb_hbm, drb_hbm, prb_hbm),
            (yrt_hbm, srt_hbm, drt_hbm, prt_hbm)):
        pltpu.sync_copy(zeros2_hbm, rows_v.at[NSL - 1])
        for z in range(KB):
            pltpu.async_copy(rows_v.at[NSL - 1],
                             acc_s.at[pl.ds(sid * RPT + z * CL, CL), :],
                             ssems[0])
        pltpu.sync_copy(s_hbm.at[cid, sid], sidx_v)
        pltpu.sync_copy(d_hbm.at[cid, sid], didx_v)

        def gfire(jn, bn, y_hbm=y_hbm):
            pltpu.async_copy(y_hbm.at[sidx_v.at[jn]], rows_v.at[bn],
                             gsems[bn])

        def gwait(j, b, y_hbm=y_hbm):
            pltpu.make_async_copy(y_hbm.at[sidx_v.at[j]], rows_v.at[b],
                                  gsems[b]).wait()

        def sfire(j, b):
            pltpu.async_copy(rows_v.at[b], acc_s.at[didx_v.at[j]], ssems[b],
                             add=True)

        def swait(j, b):
            pltpu.make_async_copy(rows_v.at[b], acc_s.at[didx_v.at[j]],
                                  ssems[b]).wait()

        gfire(0, 0)
        gfire(1, 1)
        for z in range(KB):
            pltpu.make_async_copy(
                rows_v.at[NSL - 1], acc_s.at[pl.ds(sid * RPT + z * CL, CL), :],
                ssems[0]).wait()
        plsc.subcore_barrier()
        gwait(0, 0)
        sfire(0, 0)
        gfire(2, 2)

        def step(t, carry, y_hbm=y_hbm):
            for i in range(NSL):
                j = 1 + t * NSL + i
                b = (1 + i) % NSL
                gwait(j, b)
                sfire(j, b)
                jn = j + 2
                bn = i % NSL

                @pl.when(jn < CH)
                def _(jn=jn, bn=bn):
                    swait(jn - NSL, bn)
                    gfire(jn, bn)
            return carry

        lax.fori_loop(0, (CH - 1) // NSL, step, 0)
        for j in range(CH - NSL, CH):
            swait(j, j % NSL)
        plsc.subcore_barrier()
        for k in range(KB):
            sl = k % NSL
            if k >= NSL:
                rp = sid * RPT + (k - NSL) * CL
                pltpu.make_async_copy(rows_v.at[sl],
                                      p_hbm.at[cid, pl.ds(rp, CL), :],
                                      gsems[sl]).wait()
            r0 = sid * RPT + k * CL
            pltpu.sync_copy(acc_s.at[pl.ds(r0, CL), :], rows_v.at[sl])
            pltpu.async_copy(rows_v.at[sl], p_hbm.at[cid, pl.ds(r0, CL), :],
                             gsems[sl])
        for k in range(KB - NSL, KB):
            sl = k % NSL
            r0 = sid * RPT + k * CL
            pltpu.make_async_copy(rows_v.at[sl],
                                  p_hbm.at[cid, pl.ds(r0, CL), :],
                                  gsems[sl]).wait()


def _k4_body(deg_ref, pf_ref, prb_ref, prt_ref, bf_ref, brb_ref, brt_ref,
             hu_ref, hi_ref):
    def nd(slot):
        d = deg_ref[:, slot] + deg_ref[:, 6 + slot]
        return jnp.where(d > 0, lax.rsqrt(jnp.maximum(d, 1.0)), 0.0)

    agg_f = (pf_ref[0] + pf_ref[1]) * nd(1)[:, None] + bf_ref[...][None, :]
    agg_rb = (prb_ref[0] + prb_ref[1]) * nd(3)[:, None] + brb_ref[...][None, :]
    hu_ref[...] = agg_f + agg_rb
    hi_ref[...] = ((prt_ref[0] + prt_ref[1]) * nd(5)[:, None]
                   + brt_ref[...][None, :])


def _k4_combine(degp, p_f, p_rb, p_rt, b_f, b_rb, b_rt):
    h_shape = jax.ShapeDtypeStruct((N, D), jnp.float32)
    part_spec = pl.BlockSpec((NC, _BM, D), lambda i: (0, i, 0))
    bias_spec = pl.BlockSpec((D,), lambda i: (0,))
    return pl.pallas_call(
        _k4_body,
        grid=(N // _BM,),
        in_specs=[
            pl.BlockSpec((_BM, 2 * 6), lambda i: (i, 0)),
            part_spec, part_spec, part_spec,
            bias_spec, bias_spec, bias_spec,
        ],
        out_specs=[
            pl.BlockSpec((_BM, D), lambda i: (i, 0)),
            pl.BlockSpec((_BM, D), lambda i: (i, 0)),
        ],
        out_shape=[h_shape, h_shape],
    )(degp, p_f, p_rb, p_rt, b_f, b_rb, b_rt)


def _pad_src(vec):
    pad = jnp.arange(EP - E, dtype=jnp.int32) % N
    return jnp.concatenate([vec, pad]).reshape(NC, NS, CH, CL)


def _pad_dst(vec):
    pad = N + (jnp.arange(EP - E, dtype=jnp.int32) % (NP - N))
    return jnp.concatenate([vec, pad]).reshape(NC, NS, CH, CL)


def kernel(x_user, x_item, edge_index_follows, edge_index_rates,
           edge_index_rated_by, W_follows, b_follows, W_rates, b_rates,
           W_rated_by, b_rated_by):
    s_f = _pad_src(edge_index_follows[0])
    d_f = _pad_dst(edge_index_follows[1])
    s_rb = _pad_src(edge_index_rated_by[0])
    d_rb = _pad_dst(edge_index_rated_by[1])
    s_rt = _pad_src(edge_index_rates[0])
    d_rt = _pad_dst(edge_index_rates[1])

    ones128 = jnp.ones((CL1,), jnp.float32)
    zeros1 = jnp.zeros((RPT,), jnp.float32)
    zeros2 = jnp.zeros((CL, D), jnp.float32)

    def r1(a):
        return a.reshape(NC, NS, CH1, CL1)

    degp = _k1_degrees(r1(s_f), r1(d_f), r1(s_rb), r1(d_rb), r1(s_rt),
                       r1(d_rt), ones128, zeros1)
    degt = jnp.swapaxes(degp.reshape(NC * 6, NP), 0, 1)

    corr = (jnp.arange(N, dtype=jnp.int32) < (EP - E)).astype(jnp.float32)
    corr = corr[:, None]
    y_f, y_rb, y_rt = _k2_transform(degt, corr, x_user, x_item, W_follows,
                                    W_rated_by, W_rates)

    p_f, p_rb, p_rt = _k3_scatter(y_f, y_rb, y_rt, s_f, d_f, s_rb, d_rb,
                                  s_rt, d_rt, zeros2)

    return _k4_combine(degt, p_f, p_rb, p_rt, b_follows, b_rated_by, b_rates)


# speedup vs baseline: 1.2500x; 1.0332x over previous
import functools

import jax
import jax.numpy as jnp
from jax import lax
from jax.experimental import pallas as pl
from jax.experimental.pallas import tpu as pltpu
from jax.experimental.pallas import tpu_sc as plsc

N = 10000
E = 160000
D = 128
NC = 2
NS = 16
CL = 80
CH = 64
CL1 = 128
CH1 = 40
EPT = CH * CL
EP = NC * NS * EPT
NP = 10240
RPT = NP // NS
KB = RPT // CL

_mesh = plsc.VectorSubcoreMesh(core_axis_name="c", subcore_axis_name="s")


NB1 = 4


@functools.partial(
    pl.kernel,
    mesh=_mesh,
    out_type=jax.ShapeDtypeStruct((NC, 6, NP), jnp.float32),
    scratch_types=[
        pltpu.VMEM((CH1, CL1), jnp.int32),
        pltpu.VMEM((CL1,), jnp.float32),
        pltpu.VMEM((RPT,), jnp.float32),
    ] + [pltpu.VMEM_SHARED((NP,), jnp.float32) for _ in range(6)]
      + [pltpu.SemaphoreType.DMA for _ in range(NB1)],
)
def _k1_degrees(i0, i1, i2, i3, i4, i5, ones_hbm, zeros1_hbm, degp_hbm,
                idx_v, ones_v, buf_v, h0, h1, h2, h3, h4, h5, s0, s1, s2, s3):
    cid = lax.axis_index("c")
    sid = lax.axis_index("s")
    hists = (h0, h1, h2, h3, h4, h5)
    idxs = (i0, i1, i2, i3, i4, i5)
    sems = (s0, s1, s2, s3)
    pltpu.sync_copy(ones_hbm, ones_v)
    pltpu.sync_copy(zeros1_hbm, buf_v)
    for h in hists:
        pltpu.sync_copy(buf_v, h.at[pl.ds(sid * RPT, RPT)])
    plsc.subcore_barrier()
    for a, (h, ix) in enumerate(zip(hists, idxs)):
        pltpu.sync_copy(ix.at[cid, sid], idx_v)
        for b in range(NB1 - 1):
            pltpu.async_copy(ones_v, h.at[idx_v.at[b]], sems[b], add=True)

        def step(t, carry, h=h):
            for b in range(NB1):
                j = t * NB1 + b
                jn = j + NB1 - 1
                fs = (b + NB1 - 1) % NB1

                @pl.when(jn < CH1)
                def _(jn=jn, fs=fs, h=h):
                    pltpu.async_copy(ones_v, h.at[idx_v.at[jn]], sems[fs],
                                     add=True)

                pltpu.make_async_copy(ones_v, h.at[idx_v.at[j]],
                                      sems[b]).wait()
            return carry

        lax.fori_loop(0, CH1 // NB1, step, 0)
    plsc.subcore_barrier()
    for a, h in enumerate(hists):
        pltpu.sync_copy(h.at[pl.ds(sid * RPT, RPT)], buf_v)
        pltpu.sync_copy(buf_v, degp_hbm.at[cid, a, pl.ds(sid * RPT, RPT)])


_BM = 2000


def _k2_body(deg_ref, xu_ref, xi_ref, wf_ref, wrb_ref, wrt_ref,
             yf_ref, yrb_ref, yrt_ref):
    def scaled(x_ref, slot):
        d = deg_ref[:, slot] + deg_ref[:, 6 + slot]
        ns = jnp.where(d > 0, lax.rsqrt(jnp.maximum(d, 1.0)), 0.0)
        return x_ref[...] * ns[:, None]

    yf_ref[...] = jnp.dot(scaled(xu_ref, 0), wf_ref[...],
                          preferred_element_type=jnp.float32)
    yrb_ref[...] = jnp.dot(scaled(xi_ref, 2), wrb_ref[...],
                           preferred_element_type=jnp.float32)
    yrt_ref[...] = jnp.dot(scaled(xu_ref, 4), wrt_ref[...],
                           preferred_element_type=jnp.float32)


def _k2_transform(degp, xu, xi, w_f, w_rb, w_rt):
    y_shape = jax.ShapeDtypeStruct((N, D), jnp.float32)
    return pl.pallas_call(
        _k2_body,
        grid=(N // _BM,),
        in_specs=[
            pl.BlockSpec((_BM, 2 * 6), lambda i: (i, 0)),
            pl.BlockSpec((_BM, D), lambda i: (i, 0)),
            pl.BlockSpec((_BM, D), lambda i: (i, 0)),
            pl.BlockSpec((D, D), lambda i: (0, 0)),
            pl.BlockSpec((D, D), lambda i: (0, 0)),
            pl.BlockSpec((D, D), lambda i: (0, 0)),
        ],
        out_specs=[
            pl.BlockSpec((_BM, D), lambda i: (i, 0)),
            pl.BlockSpec((_BM, D), lambda i: (i, 0)),
            pl.BlockSpec((_BM, D), lambda i: (i, 0)),
        ],
        out_shape=[y_shape, y_shape, y_shape],
    )(degp, xu, xi, w_f, w_rb, w_rt)


NSL = 3
_part_t = jax.ShapeDtypeStruct((NC, NP, D), jnp.float32)


@functools.partial(
    pl.kernel,
    mesh=_mesh,
    out_type=(_part_t, _part_t, _part_t),
    scratch_types=[
        pltpu.VMEM_SHARED((NP, D), jnp.float32),
        pltpu.VMEM((CH, CL), jnp.int32),
        pltpu.VMEM((CH, CL), jnp.int32),
        pltpu.VMEM((NSL, CL, D), jnp.float32),
    ] + [pltpu.SemaphoreType.DMA for _ in range(2 * NSL)],
)
def _k3_scatter(yf_hbm, yrb_hbm, yrt_hbm, sf_hbm, df_hbm, srb_hbm, drb_hbm,
                srt_hbm, drt_hbm, zeros2_hbm, pf_hbm, prb_hbm, prt_hbm,
                acc_s, sidx_v, didx_v, rows_v, g0, g1, g2, t0, t1, t2):
    cid = lax.axis_index("c")
    sid = lax.axis_index("s")
    gsems = (g0, g1, g2)
    ssems = (t0, t1, t2)
    for y_hbm, s_hbm, d_hbm, p_hbm in (
            (yf_hbm, sf_hbm, df_hbm, pf_hbm),
            (yrb_hbm, srb_hbm, drb_hbm, prb_hbm),
            (yrt_hbm, srt_hbm, drt_hbm, prt_hbm)):
        pltpu.sync_copy(zeros2_hbm, rows_v.at[NSL - 1])
        for z in range(KB):
            pltpu.async_copy(rows_v.at[NSL - 1],
                             acc_s.at[pl.ds(sid * RPT + z * CL, CL), :],
                             ssems[0])
        pltpu.sync_copy(s_hbm.at[cid, sid], sidx_v)
        pltpu.sync_copy(d_hbm.at[cid, sid], didx_v)

        def gfire(jn, bn, y_hbm=y_hbm):
            pltpu.async_copy(y_hbm.at[sidx_v.at[jn]], rows_v.at[bn],
                             gsems[bn])

        def gwait(j, b, y_hbm=y_hbm):
            pltpu.make_async_copy(y_hbm.at[sidx_v.at[j]], rows_v.at[b],
                                  gsems[b]).wait()

        def sfire(j, b):
            pltpu.async_copy(rows_v.at[b], acc_s.at[didx_v.at[j]], ssems[b],
                             add=True)

        def swait(j, b):
            pltpu.make_async_copy(rows_v.at[b], acc_s.at[didx_v.at[j]],
                                  ssems[b]).wait()

        gfire(0, 0)
        gfire(1, 1)
        for z in range(KB):
            pltpu.make_async_copy(
                rows_v.at[NSL - 1], acc_s.at[pl.ds(sid * RPT + z * CL, CL), :],
                ssems[0]).wait()
        plsc.subcore_barrier()
        gwait(0, 0)
        sfire(0, 0)
        gfire(2, 2)

        def step(t, carry, y_hbm=y_hbm):
            for i in range(NSL):
                j = 1 + t * NSL + i
                b = (1 + i) % NSL
                gwait(j, b)
                sfire(j, b)
                jn = j + 2
                bn = i % NSL

                @pl.when(jn < CH)
                def _(jn=jn, bn=bn):
                    swait(jn - NSL, bn)
                    gfire(jn, bn)
            return carry

        lax.fori_loop(0, (CH - 1) // NSL, step, 0)
        for j in range(CH - NSL, CH):
            swait(j, j % NSL)
        plsc.subcore_barrier()
        for k in range(KB):
            sl = k % NSL
            if k >= NSL:
                rp = sid * RPT + (k - NSL) * CL
                pltpu.make_async_copy(rows_v.at[sl],
                                      p_hbm.at[cid, pl.ds(rp, CL), :],
                                      gsems[sl]).wait()
            r0 = sid * RPT + k * CL
            pltpu.sync_copy(acc_s.at[pl.ds(r0, CL), :], rows_v.at[sl])
            pltpu.async_copy(rows_v.at[sl], p_hbm.at[cid, pl.ds(r0, CL), :],
                             gsems[sl])
        for k in range(KB - NSL, KB):
            sl = k % NSL
            r0 = sid * RPT + k * CL
            pltpu.make_async_copy(rows_v.at[sl],
                                  p_hbm.at[cid, pl.ds(r0, CL), :],
                                  gsems[sl]).wait()


def _k4_body(deg_ref, pf_ref, prb_ref, prt_ref, bf_ref, brb_ref, brt_ref,
             hu_ref, hi_ref):
    def nd(slot):
        d = deg_ref[:, slot] + deg_ref[:, 6 + slot]
        return jnp.where(d > 0, lax.rsqrt(jnp.maximum(d, 1.0)), 0.0)

    agg_f = (pf_ref[0] + pf_ref[1]) * nd(1)[:, None] + bf_ref[...][None, :]
    agg_rb = (prb_ref[0] + prb_ref[1]) * nd(3)[:, None] + brb_ref[...][None, :]
    hu_ref[...] = agg_f + agg_rb
    hi_ref[...] = ((prt_ref[0] + prt_ref[1]) * nd(5)[:, None]
                   + brt_ref[...][None, :])


def _k4_combine(degp, p_f, p_rb, p_rt, b_f, b_rb, b_rt):
    h_shape = jax.ShapeDtypeStruct((N, D), jnp.float32)
    part_spec = pl.BlockSpec((NC, _BM, D), lambda i: (0, i, 0))
    bias_spec = pl.BlockSpec((D,), lambda i: (0,))
    return pl.pallas_call(
        _k4_body,
        grid=(N // _BM,),
        in_specs=[
            pl.BlockSpec((_BM, 2 * 6), lambda i: (i, 0)),
            part_spec, part_spec, part_spec,
            bias_spec, bias_spec, bias_spec,
        ],
        out_specs=[
            pl.BlockSpec((_BM, D), lambda i: (i, 0)),
            pl.BlockSpec((_BM, D), lambda i: (i, 0)),
        ],
        out_shape=[h_shape, h_shape],
    )(degp, p_f, p_rb, p_rt, b_f, b_rb, b_rt)


def _pad_src(vec):
    pad = jnp.arange(EP - E, dtype=jnp.int32) % N
    return jnp.concatenate([vec, pad]).reshape(NC, NS, CH, CL)


def _pad_dst(vec):
    pad = N + (jnp.arange(EP - E, dtype=jnp.int32) % (NP - N))
    return jnp.concatenate([vec, pad]).reshape(NC, NS, CH, CL)


def kernel(x_user, x_item, edge_index_follows, edge_index_rates,
           edge_index_rated_by, W_follows, b_follows, W_rates, b_rates,
           W_rated_by, b_rated_by):
    s_f = _pad_src(edge_index_follows[0])
    d_f = _pad_dst(edge_index_follows[1])
    s_rb = _pad_src(edge_index_rated_by[0])
    d_rb = _pad_dst(edge_index_rated_by[1])
    s_rt = _pad_src(edge_index_rates[0])
    d_rt = _pad_dst(edge_index_rates[1])

    ones128 = jnp.ones((CL1,), jnp.float32)
    zeros1 = jnp.zeros((RPT,), jnp.float32)
    zeros2 = jnp.zeros((CL, D), jnp.float32)

    def r1(a):
        return a.reshape(NC, NS, CH1, CL1)

    sj_f = _pad_dst(edge_index_follows[0])
    sj_rb = _pad_dst(edge_index_rated_by[0])
    sj_rt = _pad_dst(edge_index_rates[0])
    degp = _k1_degrees(r1(sj_f), r1(d_f), r1(sj_rb), r1(d_rb), r1(sj_rt),
                       r1(d_rt), ones128, zeros1)
    degt = jnp.swapaxes(degp.reshape(NC * 6, NP), 0, 1)

    y_f, y_rb, y_rt = _k2_transform(degt, x_user, x_item, W_follows,
                                    W_rated_by, W_rates)

    p_f, p_rb, p_rt = _k3_scatter(y_f, y_rb, y_rt, s_f, d_f, s_rb, d_rb,
                                  s_rt, d_rt, zeros2)

    return _k4_combine(degt, p_f, p_rb, p_rt, b_follows, b_rated_by, b_rates)
